# Initial kernel scaffold; baseline (speedup 1.0000x reference)
import jax
import jax.numpy as jnp
from jax.experimental import pallas as pl


def kernel(x, W_router, ln1_g, ln1_b, Wq, Wk, Wv, Wo, ln2_g, ln2_b, Wg, Wu, Wd):
    raise NotImplementedError("write your pallas kernel here")


# speedup vs baseline: 1.4312x; 1.4312x over previous
import functools

import jax
import jax.numpy as jnp
from jax import lax
from jax.experimental import pallas as pl
from jax.experimental.pallas import tpu as pltpu
from jax.experimental.pallas import tpu_sc as plsc

H = 2048
NH = 16
DH = H // NH
FF = 8192
CAP = 1024
S = 2048
EPS = 1e-6
NEG = -1e9


def _ln(xb, g, b):
    m = jnp.mean(xb, axis=-1, keepdims=True)
    d = xb - m
    v = jnp.mean(d * d, axis=-1, keepdims=True)
    return d / jnp.sqrt(v + EPS) * g + b


def _router_body(wr_ref, x_ref, lg_ref):
    lg_ref[...] = lax.dot_general(
        wr_ref[...], x_ref[...], (((1,), (1,)), ((), ())),
        preferred_element_type=jnp.float32)


_router = pl.pallas_call(
    _router_body,
    out_shape=jax.ShapeDtypeStruct((1, S), jnp.float32),
)


def _select_body(lg_ref, g_ref, idx_ref):
    CH = 256
    v_row = lg_ref[...]
    i0 = lax.broadcasted_iota(jnp.int32, (S, CH), 0)
    v_col = jnp.zeros((S, 1), jnp.float32)
    for t in range(S // CH):
        j1 = lax.broadcasted_iota(jnp.int32, (S, CH), 1) + t * CH
        vjb = v_row[:, t * CH:(t + 1) * CH]
        eye_t = (i0 == j1).astype(jnp.float32)
        v_col = v_col + lax.dot_general(
            eye_t, vjb, (((1,), (1,)), ((), ())),
            preferred_element_type=jnp.float32)
    rank_col = jnp.zeros((S, 1), jnp.float32)
    slot_col = jnp.zeros((S, 1), jnp.float32)
    for t in range(S // CH):
        j1 = lax.broadcasted_iota(jnp.int32, (S, CH), 1) + t * CH
        vjb = v_row[:, t * CH:(t + 1) * CH]
        beats = (vjb > v_col) | ((vjb == v_col) & (j1 < i0))
        rank_col = rank_col + jnp.sum(
            jnp.where(beats, 1.0, 0.0), axis=1, keepdims=True)
        beaten = (v_col > vjb) | ((v_col == vjb) & (i0 < j1))
        rank_row_c = jnp.sum(jnp.where(beaten, 1.0, 0.0), axis=0, keepdims=True)
        m_row_c = jnp.where(rank_row_c < CAP, 1.0, 0.0)
        slot_col = slot_col + jnp.sum(
            jnp.where(j1 < i0, m_row_c * jnp.ones((S, CH), jnp.float32), 0.0),
            axis=1, keepdims=True)
    m_col = rank_col < CAP
    icol = lax.broadcasted_iota(jnp.int32, (S, 1), 0)
    sloti = slot_col.astype(jnp.int32)
    g_ref[...] = jnp.where(m_col, S + sloti, icol)
    i0f = i0.astype(jnp.float32)
    for u in range(CAP // CH):
        sc = lax.broadcasted_iota(jnp.int32, (S, CH), 1) + u * CH
        contrib = jnp.where(m_col & (sloti == sc), i0f, 0.0)
        idx_ref[:, u * CH:(u + 1) * CH] = jnp.sum(
            contrib, axis=0, keepdims=True).astype(jnp.int32)


_select = pl.pallas_call(
    _select_body,
    out_shape=(jax.ShapeDtypeStruct((S, 1), jnp.int32),
               jax.ShapeDtypeStruct((1, CAP), jnp.int32)),
)


def _kv_body(x_ref, g_ref, b_ref, wk_ref, wv_ref, k_ref, v_ref):
    xn = _ln(x_ref[...], g_ref[...], b_ref[...])
    k_ref[...] = jnp.dot(xn, wk_ref[...], preferred_element_type=jnp.float32)
    v_ref[...] = jnp.dot(xn, wv_ref[...], preferred_element_type=jnp.float32)


_kv = pl.pallas_call(
    _kv_body,
    grid=(8, 8),
    in_specs=[
        pl.BlockSpec((256, H), lambda i, j: (i, 0)),
        pl.BlockSpec((1, H), lambda i, j: (0, 0)),
        pl.BlockSpec((1, H), lambda i, j: (0, 0)),
        pl.BlockSpec((H, 256), lambda i, j: (0, j)),
        pl.BlockSpec((H, 256), lambda i, j: (0, j)),
    ],
    out_specs=[pl.BlockSpec((256, 256), lambda i, j: (i, j)),
               pl.BlockSpec((256, 256), lambda i, j: (i, j))],
    out_shape=[jax.ShapeDtypeStruct((S, H), jnp.float32),
               jax.ShapeDtypeStruct((S, H), jnp.float32)],
)


def _q_body(x_ref, g_ref, b_ref, wq_ref, q_ref):
    xn = _ln(x_ref[...], g_ref[...], b_ref[...])
    q_ref[...] = jnp.dot(xn, wq_ref[...], preferred_element_type=jnp.float32)


_q = pl.pallas_call(
    _q_body,
    grid=(4, 8),
    in_specs=[
        pl.BlockSpec((256, H), lambda i, j: (i, 0)),
        pl.BlockSpec((1, H), lambda i, j: (0, 0)),
        pl.BlockSpec((1, H), lambda i, j: (0, 0)),
        pl.BlockSpec((H, 256), lambda i, j: (0, j)),
    ],
    out_specs=pl.BlockSpec((256, 256), lambda i, j: (i, j)),
    out_shape=jax.ShapeDtypeStruct((CAP, H), jnp.float32),
)


def _attn_body(q_ref, k_ref, v_ref, pos_ref, o_ref):
    s = lax.dot_general(
        q_ref[...], k_ref[...], (((1,), (1,)), ((), ())),
        preferred_element_type=jnp.float32) * (DH ** -0.5)
    kj = lax.broadcasted_iota(jnp.int32, (256, S), 1)
    s = jnp.where(kj <= pos_ref[...], s, NEG)
    m = jnp.max(s, axis=1, keepdims=True)
    e = jnp.exp(s - m)
    p = e / jnp.sum(e, axis=1, keepdims=True)
    o_ref[...] = jnp.dot(p, v_ref[...], preferred_element_type=jnp.float32)


_attn = pl.pallas_call(
    _attn_body,
    grid=(NH, 4),
    in_specs=[
        pl.BlockSpec((256, DH), lambda h, i: (i, h)),
        pl.BlockSpec((S, DH), lambda h, i: (0, h)),
        pl.BlockSpec((S, DH), lambda h, i: (0, h)),
        pl.BlockSpec((256, 1), lambda h, i: (i, 0)),
    ],
    out_specs=pl.BlockSpec((256, DH), lambda h, i: (i, h)),
    out_shape=jax.ShapeDtypeStruct((CAP, H), jnp.float32),
)


def _oproj_body(ao_ref, wo_ref, xs_ref, h_ref):
    h_ref[...] = xs_ref[...] + jnp.dot(
        ao_ref[...], wo_ref[...], preferred_element_type=jnp.float32)


_oproj = pl.pallas_call(
    _oproj_body,
    grid=(4, 8),
    in_specs=[
        pl.BlockSpec((256, H), lambda i, j: (i, 0)),
        pl.BlockSpec((H, 256), lambda i, j: (0, j)),
        pl.BlockSpec((256, 256), lambda i, j: (i, j)),
    ],
    out_specs=pl.BlockSpec((256, 256), lambda i, j: (i, j)),
    out_shape=jax.ShapeDtypeStruct((CAP, H), jnp.float32),
)


def _ln2_body(h_ref, g_ref, b_ref, xn_ref):
    xn_ref[...] = _ln(h_ref[...], g_ref[...], b_ref[...])


_ln2k = pl.pallas_call(
    _ln2_body,
    grid=(4,),
    in_specs=[
        pl.BlockSpec((256, H), lambda i: (i, 0)),
        pl.BlockSpec((1, H), lambda i: (0, 0)),
        pl.BlockSpec((1, H), lambda i: (0, 0)),
    ],
    out_specs=pl.BlockSpec((256, H), lambda i: (i, 0)),
    out_shape=jax.ShapeDtypeStruct((CAP, H), jnp.float32),
)


def _ffn_body(xn_ref, h_ref, wg_ref, wu_ref, wd_ref, y_ref):
    j = pl.program_id(1)
    xn = xn_ref[...]
    g = jnp.dot(xn, wg_ref[...], preferred_element_type=jnp.float32)
    u = jnp.dot(xn, wu_ref[...], preferred_element_type=jnp.float32)
    f = g / (1.0 + jnp.exp(-g)) * u
    c = jnp.dot(f, wd_ref[...], preferred_element_type=jnp.float32)

    @pl.when(j == 0)
    def _():
        y_ref[...] = h_ref[...] + c

    @pl.when(j != 0)
    def _():
        y_ref[...] = y_ref[...] + c


_ffn = pl.pallas_call(
    _ffn_body,
    grid=(2, 16),
    in_specs=[
        pl.BlockSpec((512, H), lambda i, j: (i, 0)),
        pl.BlockSpec((512, H), lambda i, j: (i, 0)),
        pl.BlockSpec((H, 512), lambda i, j: (0, j)),
        pl.BlockSpec((H, 512), lambda i, j: (0, j)),
        pl.BlockSpec((512, H), lambda i, j: (j, 0)),
    ],
    out_specs=pl.BlockSpec((512, H), lambda i, j: (i, 0)),
    out_shape=jax.ShapeDtypeStruct((CAP, H), jnp.float32),
)


@functools.lru_cache(maxsize=None)
def _sc_gather(nrows, nidx, chunk=32):
    info = plsc.get_sparse_core_info()
    nc, ns = info.num_cores, info.num_subcores
    nw = nc * ns
    b_per_w = nidx // nw
    nchunks = b_per_w // chunk
    mesh = plsc.VectorSubcoreMesh(core_axis_name="c", subcore_axis_name="s")

    @functools.partial(
        pl.kernel, mesh=mesh,
        out_type=jax.ShapeDtypeStruct((nidx, H), jnp.float32),
        scratch_types=[
            pltpu.VMEM((chunk,), jnp.int32),
            pltpu.VMEM((chunk, H), jnp.float32),
            pltpu.SemaphoreType.DMA,
        ],
    )
    def k(table_hbm, idx_hbm, out_hbm, idx_v, rows_v, sem):
        wid = lax.axis_index("s") * nc + lax.axis_index("c")
        base = wid * b_per_w
        for t in range(nchunks):
            off = base + t * chunk
            pltpu.sync_copy(idx_hbm.at[pl.ds(off, chunk)], idx_v)
            pltpu.async_copy(table_hbm.at[idx_v], rows_v, sem).wait()
            pltpu.sync_copy(rows_v, out_hbm.at[pl.ds(off, chunk)])

    return k


def kernel(x, W_router, ln1_g, ln1_b, Wq, Wk, Wv, Wo, ln2_g, ln2_b, Wg, Wu, Wd):
    B = x.shape[0]
    x2 = x.reshape(S, H)
    wr_row = W_router.reshape(1, H)
    g1 = ln1_g.reshape(1, H)
    b1 = ln1_b.reshape(1, H)
    g2 = ln2_g.reshape(1, H)
    b2 = ln2_b.reshape(1, H)

    logits_row = _router(wr_row, x2)
    gidx_col, sel_row = _select(logits_row)
    sel = sel_row.reshape(CAP)
    gidx = gidx_col.reshape(S)
    pos_col = sel_row.reshape(CAP, 1)

    x_sel = _sc_gather(S, CAP)(x2, sel)
    k_all, v_all = _kv(x2, g1, b1, Wk, Wv)
    q = _q(x_sel, g1, b1, Wq)
    ao = _attn(q, k_all, v_all, pos_col)
    h_sel = _oproj(ao, Wo, x_sel)
    xn2 = _ln2k(h_sel, g2, b2)
    y_sel = _ffn(xn2, h_sel, Wg, Wu, Wd)

    table = jnp.concatenate([x2, y_sel], axis=0)
    out2 = _sc_gather(S + CAP, S)(table, gidx)
    return (out2.reshape(B, S, H), logits_row.reshape(B, S))


# speedup vs baseline: 1.4578x; 1.0186x over previous
import functools

import jax
import jax.numpy as jnp
from jax import lax
from jax.experimental import pallas as pl
from jax.experimental.pallas import tpu as pltpu
from jax.experimental.pallas import tpu_sc as plsc

H = 2048
NH = 16
DH = H // NH
FF = 8192
CAP = 1024
S = 2048
EPS = 1e-6
NEG = -1e9


def _ln(xb, g, b):
    m = jnp.mean(xb, axis=-1, keepdims=True)
    d = xb - m
    v = jnp.mean(d * d, axis=-1, keepdims=True)
    return d / jnp.sqrt(v + EPS) * g + b


def _router_body(wr_ref, x_ref, lg_ref):
    lg_ref[...] = lax.dot_general(
        wr_ref[...], x_ref[...], (((1,), (1,)), ((), ())),
        preferred_element_type=jnp.float32)


_router = pl.pallas_call(
    _router_body,
    out_shape=jax.ShapeDtypeStruct((1, S), jnp.float32),
)


def _select_body(lg_ref, g_ref, idx_ref):
    CH = 256
    v_row = lg_ref[...]
    i0 = lax.broadcasted_iota(jnp.int32, (S, CH), 0)
    v_col = jnp.zeros((S, 1), jnp.float32)
    for t in range(S // CH):
        j1 = lax.broadcasted_iota(jnp.int32, (S, CH), 1) + t * CH
        vjb = v_row[:, t * CH:(t + 1) * CH]
        eye_t = (i0 == j1).astype(jnp.float32)
        v_col = v_col + lax.dot_general(
            eye_t, vjb, (((1,), (1,)), ((), ())),
            preferred_element_type=jnp.float32)
    rank_col = jnp.zeros((S, 1), jnp.float32)
    slot_col = jnp.zeros((S, 1), jnp.float32)
    for t in range(S // CH):
        j1 = lax.broadcasted_iota(jnp.int32, (S, CH), 1) + t * CH
        vjb = v_row[:, t * CH:(t + 1) * CH]
        beats = (vjb > v_col) | ((vjb == v_col) & (j1 < i0))
        rank_col = rank_col + jnp.sum(
            jnp.where(beats, 1.0, 0.0), axis=1, keepdims=True)
        beaten = (v_col > vjb) | ((v_col == vjb) & (i0 < j1))
        rank_row_c = jnp.sum(jnp.where(beaten, 1.0, 0.0), axis=0, keepdims=True)
        m_row_c = jnp.where(rank_row_c < CAP, 1.0, 0.0)
        slot_col = slot_col + jnp.sum(
            jnp.where(j1 < i0, m_row_c * jnp.ones((S, CH), jnp.float32), 0.0),
            axis=1, keepdims=True)
    m_col = rank_col < CAP
    icol = lax.broadcasted_iota(jnp.int32, (S, 1), 0)
    sloti = slot_col.astype(jnp.int32)
    g_ref[...] = jnp.where(m_col, S + sloti, icol)
    i0f = i0.astype(jnp.float32)
    for u in range(CAP // CH):
        sc = lax.broadcasted_iota(jnp.int32, (S, CH), 1) + u * CH
        contrib = jnp.where(m_col & (sloti == sc), i0f, 0.0)
        idx_ref[:, u * CH:(u + 1) * CH] = jnp.sum(
            contrib, axis=0, keepdims=True).astype(jnp.int32)


_select = pl.pallas_call(
    _select_body,
    out_shape=(jax.ShapeDtypeStruct((S, 1), jnp.int32),
               jax.ShapeDtypeStruct((1, CAP), jnp.int32)),
)


def _kv_body(x_ref, g_ref, b_ref, wk_ref, wv_ref, k_ref, v_ref):
    xn = _ln(x_ref[...], g_ref[...], b_ref[...]).astype(jnp.bfloat16)
    k_ref[...] = jnp.dot(xn, wk_ref[...],
                         preferred_element_type=jnp.float32).astype(jnp.bfloat16)
    v_ref[...] = jnp.dot(xn, wv_ref[...],
                         preferred_element_type=jnp.float32).astype(jnp.bfloat16)


_kv = pl.pallas_call(
    _kv_body,
    grid=(8, 4),
    in_specs=[
        pl.BlockSpec((256, H), lambda i, j: (i, 0)),
        pl.BlockSpec((1, H), lambda i, j: (0, 0)),
        pl.BlockSpec((1, H), lambda i, j: (0, 0)),
        pl.BlockSpec((H, 512), lambda i, j: (0, j)),
        pl.BlockSpec((H, 512), lambda i, j: (0, j)),
    ],
    out_specs=[pl.BlockSpec((256, 512), lambda i, j: (i, j)),
               pl.BlockSpec((256, 512), lambda i, j: (i, j))],
    out_shape=[jax.ShapeDtypeStruct((S, H), jnp.bfloat16),
               jax.ShapeDtypeStruct((S, H), jnp.bfloat16)],
)


def _q_body(x_ref, g_ref, b_ref, wq_ref, q_ref):
    xn = _ln(x_ref[...], g_ref[...], b_ref[...]).astype(jnp.bfloat16)
    q_ref[...] = jnp.dot(xn, wq_ref[...],
                         preferred_element_type=jnp.float32).astype(jnp.bfloat16)


_q = pl.pallas_call(
    _q_body,
    grid=(4, 4),
    in_specs=[
        pl.BlockSpec((256, H), lambda i, j: (i, 0)),
        pl.BlockSpec((1, H), lambda i, j: (0, 0)),
        pl.BlockSpec((1, H), lambda i, j: (0, 0)),
        pl.BlockSpec((H, 512), lambda i, j: (0, j)),
    ],
    out_specs=pl.BlockSpec((256, 512), lambda i, j: (i, j)),
    out_shape=jax.ShapeDtypeStruct((CAP, H), jnp.bfloat16),
)


def _attn_body(q_ref, k_ref, v_ref, pos_ref, o_ref):
    s = lax.dot_general(
        q_ref[...], k_ref[...], (((1,), (1,)), ((), ())),
        preferred_element_type=jnp.float32) * (DH ** -0.5)
    kj = lax.broadcasted_iota(jnp.int32, (256, S), 1)
    s = jnp.where(kj <= pos_ref[...], s, NEG)
    m = jnp.max(s, axis=1, keepdims=True)
    e = jnp.exp(s - m)
    p = (e / jnp.sum(e, axis=1, keepdims=True)).astype(jnp.bfloat16)
    o_ref[...] = jnp.dot(p, v_ref[...],
                         preferred_element_type=jnp.float32).astype(jnp.bfloat16)


_attn = pl.pallas_call(
    _attn_body,
    grid=(NH, 4),
    in_specs=[
        pl.BlockSpec((256, DH), lambda h, i: (i, h)),
        pl.BlockSpec((S, DH), lambda h, i: (0, h)),
        pl.BlockSpec((S, DH), lambda h, i: (0, h)),
        pl.BlockSpec((256, 1), lambda h, i: (i, 0)),
    ],
    out_specs=pl.BlockSpec((256, DH), lambda h, i: (i, h)),
    out_shape=jax.ShapeDtypeStruct((CAP, H), jnp.bfloat16),
)


def _oproj_body(ao_ref, wo_ref, xs_ref, h_ref):
    h_ref[...] = xs_ref[...] + jnp.dot(
        ao_ref[...], wo_ref[...], preferred_element_type=jnp.float32)


_oproj = pl.pallas_call(
    _oproj_body,
    grid=(4, 4),
    in_specs=[
        pl.BlockSpec((256, H), lambda i, j: (i, 0)),
        pl.BlockSpec((H, 512), lambda i, j: (0, j)),
        pl.BlockSpec((256, 512), lambda i, j: (i, j)),
    ],
    out_specs=pl.BlockSpec((256, 512), lambda i, j: (i, j)),
    out_shape=jax.ShapeDtypeStruct((CAP, H), jnp.float32),
)


def _ln2_body(h_ref, g_ref, b_ref, xn_ref):
    xn_ref[...] = _ln(h_ref[...], g_ref[...], b_ref[...]).astype(jnp.bfloat16)


_ln2k = pl.pallas_call(
    _ln2_body,
    grid=(4,),
    in_specs=[
        pl.BlockSpec((256, H), lambda i: (i, 0)),
        pl.BlockSpec((1, H), lambda i: (0, 0)),
        pl.BlockSpec((1, H), lambda i: (0, 0)),
    ],
    out_specs=pl.BlockSpec((256, H), lambda i: (i, 0)),
    out_shape=jax.ShapeDtypeStruct((CAP, H), jnp.bfloat16),
)


def _ffn_body(xn_ref, h_ref, wg_ref, wu_ref, wd_ref, y_ref):
    j = pl.program_id(1)
    xn = xn_ref[...]
    g = jnp.dot(xn, wg_ref[...], preferred_element_type=jnp.float32)
    u = jnp.dot(xn, wu_ref[...], preferred_element_type=jnp.float32)
    f = (g / (1.0 + jnp.exp(-g)) * u).astype(jnp.bfloat16)
    c = jnp.dot(f, wd_ref[...], preferred_element_type=jnp.float32)

    @pl.when(j == 0)
    def _():
        y_ref[...] = h_ref[...] + c

    @pl.when(j != 0)
    def _():
        y_ref[...] = y_ref[...] + c


_ffn = pl.pallas_call(
    _ffn_body,
    grid=(2, 8),
    in_specs=[
        pl.BlockSpec((512, H), lambda i, j: (i, 0)),
        pl.BlockSpec((512, H), lambda i, j: (i, 0)),
        pl.BlockSpec((H, 1024), lambda i, j: (0, j)),
        pl.BlockSpec((H, 1024), lambda i, j: (0, j)),
        pl.BlockSpec((1024, H), lambda i, j: (j, 0)),
    ],
    out_specs=pl.BlockSpec((512, H), lambda i, j: (i, 0)),
    out_shape=jax.ShapeDtypeStruct((CAP, H), jnp.float32),
)


@functools.lru_cache(maxsize=None)
def _sc_gather(nrows, nidx, chunk=32):
    info = plsc.get_sparse_core_info()
    nc, ns = info.num_cores, info.num_subcores
    nw = nc * ns
    b_per_w = nidx // nw
    nchunks = b_per_w // chunk
    mesh = plsc.VectorSubcoreMesh(core_axis_name="c", subcore_axis_name="s")

    @functools.partial(
        pl.kernel, mesh=mesh,
        out_type=jax.ShapeDtypeStruct((nidx, H), jnp.float32),
        scratch_types=[
            pltpu.VMEM((chunk,), jnp.int32),
            pltpu.VMEM((chunk, H), jnp.float32),
            pltpu.SemaphoreType.DMA,
        ],
    )
    def k(table_hbm, idx_hbm, out_hbm, idx_v, rows_v, sem):
        wid = lax.axis_index("s") * nc + lax.axis_index("c")
        base = wid * b_per_w
        for t in range(nchunks):
            off = base + t * chunk
            pltpu.sync_copy(idx_hbm.at[pl.ds(off, chunk)], idx_v)
            pltpu.async_copy(table_hbm.at[idx_v], rows_v, sem).wait()
            pltpu.sync_copy(rows_v, out_hbm.at[pl.ds(off, chunk)])

    return k


def kernel(x, W_router, ln1_g, ln1_b, Wq, Wk, Wv, Wo, ln2_g, ln2_b, Wg, Wu, Wd):
    B = x.shape[0]
    x2 = x.reshape(S, H)
    wr_row = W_router.reshape(1, H)
    g1 = ln1_g.reshape(1, H)
    b1 = ln1_b.reshape(1, H)
    g2 = ln2_g.reshape(1, H)
    b2 = ln2_b.reshape(1, H)

    bf = jnp.bfloat16
    wq, wk, wv, wo = Wq.astype(bf), Wk.astype(bf), Wv.astype(bf), Wo.astype(bf)
    wg, wu, wd = Wg.astype(bf), Wu.astype(bf), Wd.astype(bf)

    logits_row = _router(wr_row, x2)
    gidx_col, sel_row = _select(logits_row)
    sel = sel_row.reshape(CAP)
    gidx = gidx_col.reshape(S)
    pos_col = sel_row.reshape(CAP, 1)

    x_sel = _sc_gather(S, CAP)(x2, sel)
    k_all, v_all = _kv(x2, g1, b1, wk, wv)
    q = _q(x_sel, g1, b1, wq)
    ao = _attn(q, k_all, v_all, pos_col)
    h_sel = _oproj(ao, wo, x_sel)
    xn2 = _ln2k(h_sel, g2, b2)
    y_sel = _ffn(xn2, h_sel, wg, wu, wd)

    table = jnp.concatenate([x2, y_sel], axis=0)
    out2 = _sc_gather(S + CAP, S)(table, gidx)
    return (out2.reshape(B, S, H), logits_row.reshape(B, S))


# speedup vs baseline: 1.4674x; 1.0066x over previous
import functools

import jax
import jax.numpy as jnp
from jax import lax
from jax.experimental import pallas as pl
from jax.experimental.pallas import tpu as pltpu
from jax.experimental.pallas import tpu_sc as plsc

H = 2048
NH = 16
DH = H // NH
FF = 8192
CAP = 1024
S = 2048
EPS = 1e-6
NEG = -1e9
BF = jnp.bfloat16


def _ln(xb, g, b):
    m = jnp.mean(xb, axis=-1, keepdims=True)
    d = xb - m
    v = jnp.mean(d * d, axis=-1, keepdims=True)
    return d / jnp.sqrt(v + EPS) * g + b


def _route_body(x_ref, wr_ref, g1_ref, b1_ref, lg_ref, xn_ref, g_ref, idx_ref):
    x = x_ref[...]
    xn_ref[...] = _ln(x, g1_ref[...], b1_ref[...]).astype(BF)
    v_row = lax.dot_general(wr_ref[...], x, (((1,), (1,)), ((), ())),
                            preferred_element_type=jnp.float32)
    lg_ref[...] = v_row
    CH = 256
    i0 = lax.broadcasted_iota(jnp.int32, (S, CH), 0)
    v_col = jnp.zeros((S, 1), jnp.float32)
    for t in range(S // CH):
        j1 = lax.broadcasted_iota(jnp.int32, (S, CH), 1) + t * CH
        vjb = v_row[:, t * CH:(t + 1) * CH]
        eye_t = (i0 == j1).astype(jnp.float32)
        v_col = v_col + lax.dot_general(
            eye_t, vjb, (((1,), (1,)), ((), ())),
            preferred_element_type=jnp.float32)
    rank_col = jnp.zeros((S, 1), jnp.float32)
    slot_col = jnp.zeros((S, 1), jnp.float32)
    for t in range(S // CH):
        j1 = lax.broadcasted_iota(jnp.int32, (S, CH), 1) + t * CH
        vjb = v_row[:, t * CH:(t + 1) * CH]
        beats = (vjb > v_col) | ((vjb == v_col) & (j1 < i0))
        rank_col = rank_col + jnp.sum(
            jnp.where(beats, 1.0, 0.0), axis=1, keepdims=True)
        beaten = (v_col > vjb) | ((v_col == vjb) & (i0 < j1))
        rank_row_c = jnp.sum(jnp.where(beaten, 1.0, 0.0), axis=0, keepdims=True)
        m_row_c = jnp.where(rank_row_c < CAP, 1.0, 0.0)
        slot_col = slot_col + jnp.sum(
            jnp.where(j1 < i0, m_row_c * jnp.ones((S, CH), jnp.float32), 0.0),
            axis=1, keepdims=True)
    m_col = rank_col < CAP
    icol = lax.broadcasted_iota(jnp.int32, (S, 1), 0)
    sloti = slot_col.astype(jnp.int32)
    g_ref[...] = jnp.where(m_col, S + sloti, icol)
    i0f = i0.astype(jnp.float32)
    for u in range(CAP // CH):
        sc = lax.broadcasted_iota(jnp.int32, (S, CH), 1) + u * CH
        contrib = jnp.where(m_col & (sloti == sc), i0f, 0.0)
        idx_ref[:, u * CH:(u + 1) * CH] = jnp.sum(
            contrib, axis=0, keepdims=True).astype(jnp.int32)


_route = pl.pallas_call(
    _route_body,
    out_shape=(jax.ShapeDtypeStruct((1, S), jnp.float32),
               jax.ShapeDtypeStruct((S, H), BF),
               jax.ShapeDtypeStruct((S, 1), jnp.int32),
               jax.ShapeDtypeStruct((1, CAP), jnp.int32)),
)


def _kv_body(xn_ref, wk_ref, wv_ref, k_ref, v_ref):
    xn = xn_ref[...]
    k_ref[...] = jnp.dot(xn, wk_ref[...].astype(BF),
                         preferred_element_type=jnp.float32).astype(BF)
    v_ref[...] = jnp.dot(xn, wv_ref[...].astype(BF),
                         preferred_element_type=jnp.float32).astype(BF)


_kv = pl.pallas_call(
    _kv_body,
    grid=(4, 8),
    in_specs=[
        pl.BlockSpec((256, H), lambda j, i: (i, 0)),
        pl.BlockSpec((H, 512), lambda j, i: (0, j)),
        pl.BlockSpec((H, 512), lambda j, i: (0, j)),
    ],
    out_specs=[pl.BlockSpec((256, 512), lambda j, i: (i, j)),
               pl.BlockSpec((256, 512), lambda j, i: (i, j))],
    out_shape=[jax.ShapeDtypeStruct((S, H), BF),
               jax.ShapeDtypeStruct((S, H), BF)],
)


def _q_body(xn_ref, wq_ref, q_ref):
    q_ref[...] = jnp.dot(xn_ref[...], wq_ref[...].astype(BF),
                         preferred_element_type=jnp.float32).astype(BF)


_q = pl.pallas_call(
    _q_body,
    grid=(4,),
    in_specs=[
        pl.BlockSpec((CAP, H), lambda j: (0, 0)),
        pl.BlockSpec((H, 512), lambda j: (0, j)),
    ],
    out_specs=pl.BlockSpec((CAP, 512), lambda j: (0, j)),
    out_shape=jax.ShapeDtypeStruct((CAP, H), BF),
)


def _attn_body(q_ref, k_ref, v_ref, pos_ref, o_ref):
    s = lax.dot_general(
        q_ref[...], k_ref[...], (((1,), (1,)), ((), ())),
        preferred_element_type=jnp.float32) * (DH ** -0.5)
    kj = lax.broadcasted_iota(jnp.int32, (256, S), 1)
    s = jnp.where(kj <= pos_ref[...], s, NEG)
    m = jnp.max(s, axis=1, keepdims=True)
    e = jnp.exp(s - m)
    p = (e / jnp.sum(e, axis=1, keepdims=True)).astype(BF)
    o_ref[...] = jnp.dot(p, v_ref[...],
                         preferred_element_type=jnp.float32).astype(BF)


_attn = pl.pallas_call(
    _attn_body,
    grid=(NH, 4),
    in_specs=[
        pl.BlockSpec((256, DH), lambda h, i: (i, h)),
        pl.BlockSpec((S, DH), lambda h, i: (0, h)),
        pl.BlockSpec((S, DH), lambda h, i: (0, h)),
        pl.BlockSpec((256, 1), lambda h, i: (i, 0)),
    ],
    out_specs=pl.BlockSpec((256, DH), lambda h, i: (i, h)),
    out_shape=jax.ShapeDtypeStruct((CAP, H), BF),
)


def _oproj_body(ao_ref, wo_ref, xs_ref, g2_ref, b2_ref, h_ref, xn_ref):
    hb = xs_ref[...] + jnp.dot(ao_ref[...], wo_ref[...].astype(BF),
                               preferred_element_type=jnp.float32)
    h_ref[...] = hb
    xn_ref[...] = _ln(hb, g2_ref[...], b2_ref[...]).astype(BF)


_oproj = pl.pallas_call(
    _oproj_body,
    grid=(4,),
    in_specs=[
        pl.BlockSpec((256, H), lambda i: (i, 0)),
        pl.BlockSpec((H, H), lambda i: (0, 0)),
        pl.BlockSpec((256, H), lambda i: (i, 0)),
        pl.BlockSpec((1, H), lambda i: (0, 0)),
        pl.BlockSpec((1, H), lambda i: (0, 0)),
    ],
    out_specs=[pl.BlockSpec((256, H), lambda i: (i, 0)),
               pl.BlockSpec((256, H), lambda i: (i, 0))],
    out_shape=[jax.ShapeDtypeStruct((CAP, H), jnp.float32),
               jax.ShapeDtypeStruct((CAP, H), BF)],
)


def _ffn_body(xn_ref, h_ref, wg_ref, wu_ref, wd_ref, y_ref):
    j = pl.program_id(0)
    xn = xn_ref[...]
    g = jnp.dot(xn, wg_ref[...].astype(BF), preferred_element_type=jnp.float32)
    u = jnp.dot(xn, wu_ref[...].astype(BF), preferred_element_type=jnp.float32)
    f = (g / (1.0 + jnp.exp(-g)) * u).astype(BF)
    c = jnp.dot(f, wd_ref[...].astype(BF), preferred_element_type=jnp.float32)

    @pl.when(j == 0)
    def _():
        y_ref[...] = h_ref[...] + c

    @pl.when(j != 0)
    def _():
        y_ref[...] = y_ref[...] + c


_ffn = pl.pallas_call(
    _ffn_body,
    grid=(32,),
    in_specs=[
        pl.BlockSpec((CAP, H), lambda j: (0, 0)),
        pl.BlockSpec((CAP, H), lambda j: (0, 0)),
        pl.BlockSpec((H, 256), lambda j: (0, j)),
        pl.BlockSpec((H, 256), lambda j: (0, j)),
        pl.BlockSpec((256, H), lambda j: (j, 0)),
    ],
    out_specs=pl.BlockSpec((CAP, H), lambda j: (0, 0)),
    out_shape=jax.ShapeDtypeStruct((CAP, H), jnp.float32),
)


@functools.lru_cache(maxsize=None)
def _sc_gather2(chunk=32):
    info = plsc.get_sparse_core_info()
    nc, ns = info.num_cores, info.num_subcores
    b_per_w = CAP // (nc * ns)
    mesh = plsc.VectorSubcoreMesh(core_axis_name="c", subcore_axis_name="s")

    @functools.partial(
        pl.kernel, mesh=mesh,
        out_type=(jax.ShapeDtypeStruct((CAP, H), jnp.float32),
                  jax.ShapeDtypeStruct((CAP, H // 2), jnp.int32)),
        scratch_types=[
            pltpu.VMEM((chunk,), jnp.int32),
            pltpu.VMEM((chunk, H), jnp.float32),
            pltpu.VMEM((chunk, H // 2), jnp.int32),
            pltpu.SemaphoreType.DMA,
        ],
    )
    def k(x_hbm, xn_hbm, idx_hbm, xs_hbm, xns_hbm, idx_v, rf_v, rb_v, sem):
        wid = lax.axis_index("s") * nc + lax.axis_index("c")
        base = wid * b_per_w
        for t in range(b_per_w // chunk):
            off = base + t * chunk
            pltpu.sync_copy(idx_hbm.at[pl.ds(off, chunk)], idx_v)
            pltpu.async_copy(x_hbm.at[idx_v], rf_v, sem).wait()
            pltpu.sync_copy(rf_v, xs_hbm.at[pl.ds(off, chunk)])
            pltpu.async_copy(xn_hbm.at[idx_v], rb_v, sem).wait()
            pltpu.sync_copy(rb_v, xns_hbm.at[pl.ds(off, chunk)])

    return k


@functools.lru_cache(maxsize=None)
def _sc_gather_out(chunk=32):
    info = plsc.get_sparse_core_info()
    nc, ns = info.num_cores, info.num_subcores
    b_per_w = S // (nc * ns)
    mesh = plsc.VectorSubcoreMesh(core_axis_name="c", subcore_axis_name="s")

    @functools.partial(
        pl.kernel, mesh=mesh,
        out_type=jax.ShapeDtypeStruct((S, H), jnp.float32),
        scratch_types=[
            pltpu.VMEM((chunk,), jnp.int32),
            pltpu.VMEM((chunk, H), jnp.float32),
            pltpu.SemaphoreType.DMA,
        ],
    )
    def k(table_hbm, idx_hbm, out_hbm, idx_v, rows_v, sem):
        wid = lax.axis_index("s") * nc + lax.axis_index("c")
        base = wid * b_per_w
        for t in range(b_per_w // chunk):
            off = base + t * chunk
            pltpu.sync_copy(idx_hbm.at[pl.ds(off, chunk)], idx_v)
            pltpu.async_copy(table_hbm.at[idx_v], rows_v, sem).wait()
            pltpu.sync_copy(rows_v, out_hbm.at[pl.ds(off, chunk)])

    return k


def kernel(x, W_router, ln1_g, ln1_b, Wq, Wk, Wv, Wo, ln2_g, ln2_b, Wg, Wu, Wd):
    B = x.shape[0]
    x2 = x.reshape(S, H)
    wr_row = W_router.reshape(1, H)
    g1 = ln1_g.reshape(1, H)
    b1 = ln1_b.reshape(1, H)
    g2 = ln2_g.reshape(1, H)
    b2 = ln2_b.reshape(1, H)

    logits_row, xn, gidx_col, sel_row = _route(x2, wr_row, g1, b1)
    sel = sel_row.reshape(CAP)
    gidx = gidx_col.reshape(S)
    pos_col = sel_row.reshape(CAP, 1)

    xn_i32 = lax.bitcast_convert_type(
        xn.reshape(S, H // 2, 2), jnp.int32)
    x_sel, xn_sel_i32 = _sc_gather2()(x2, xn_i32, sel)
    xn_sel = lax.bitcast_convert_type(
        xn_sel_i32, BF).reshape(CAP, H)
    k_all, v_all = _kv(xn, Wk, Wv)
    q = _q(xn_sel, Wq)
    ao = _attn(q, k_all, v_all, pos_col)
    h_sel, xn2 = _oproj(ao, Wo, x_sel, g2, b2)
    y_sel = _ffn(xn2, h_sel, Wg, Wu, Wd)

    table = jnp.concatenate([x2, y_sel], axis=0)
    out2 = _sc_gather_out()(table, gidx)
    return (out2.reshape(B, S, H), logits_row.reshape(B, S))


# speedup vs baseline: 1.9865x; 1.3537x over previous
import functools

import jax
import jax.numpy as jnp
from jax import lax
from jax.experimental import pallas as pl
from jax.experimental.pallas import tpu as pltpu
from jax.experimental.pallas import tpu_sc as plsc

H = 2048
NH = 16
DH = H // NH
FF = 8192
CAP = 1024
S = 2048
EPS = 1e-6
NEG = -1e9
BF = jnp.bfloat16


def _ln(xb, g, b):
    m = jnp.mean(xb, axis=-1, keepdims=True)
    d = xb - m
    v = jnp.mean(d * d, axis=-1, keepdims=True)
    return d / jnp.sqrt(v + EPS) * g + b


def _route_body(x_ref, wr_ref, g1_ref, b1_ref, lg_ref, xn_ref, g_ref, idx_ref,
                posf_ref):
    x = x_ref[...]
    xn_ref[...] = _ln(x, g1_ref[...], b1_ref[...])
    v_row = lax.dot_general(wr_ref[...], x, (((1,), (1,)), ((), ())),
                            preferred_element_type=jnp.float32)
    lg_ref[...] = v_row
    CH = 256
    i0 = lax.broadcasted_iota(jnp.int32, (S, CH), 0)
    v_col = jnp.zeros((S, 1), jnp.float32)
    for t in range(S // CH):
        j1 = lax.broadcasted_iota(jnp.int32, (S, CH), 1) + t * CH
        vjb = v_row[:, t * CH:(t + 1) * CH]
        eye_t = (i0 == j1).astype(jnp.float32)
        v_col = v_col + lax.dot_general(
            eye_t, vjb, (((1,), (1,)), ((), ())),
            preferred_element_type=jnp.float32)
    rank_col = jnp.zeros((S, 1), jnp.float32)
    slot_col = jnp.zeros((S, 1), jnp.float32)
    for t in range(S // CH):
        j1 = lax.broadcasted_iota(jnp.int32, (S, CH), 1) + t * CH
        vjb = v_row[:, t * CH:(t + 1) * CH]
        beats = (vjb > v_col) | ((vjb == v_col) & (j1 < i0))
        rank_col = rank_col + jnp.sum(
            jnp.where(beats, 1.0, 0.0), axis=1, keepdims=True)
        beaten = (v_col > vjb) | ((v_col == vjb) & (i0 < j1))
        rank_row_c = jnp.sum(jnp.where(beaten, 1.0, 0.0), axis=0, keepdims=True)
        m_row_c = jnp.where(rank_row_c < CAP, 1.0, 0.0)
        slot_col = slot_col + jnp.sum(
            jnp.where(j1 < i0, m_row_c * jnp.ones((S, CH), jnp.float32), 0.0),
            axis=1, keepdims=True)
    m_col = rank_col < CAP
    icol = lax.broadcasted_iota(jnp.int32, (S, 1), 0)
    sloti = slot_col.astype(jnp.int32)
    g_ref[...] = jnp.where(m_col, S + sloti, icol)
    i0f = i0.astype(jnp.float32)
    for u in range(CAP // CH):
        sc = lax.broadcasted_iota(jnp.int32, (S, CH), 1) + u * CH
        contrib = jnp.where(m_col & (sloti == sc), i0f, 0.0)
        idx_f = jnp.sum(contrib, axis=0, keepdims=True)
        posf_ref[:, u * CH:(u + 1) * CH] = idx_f
        idx_ref[:, u * CH:(u + 1) * CH] = idx_f.astype(jnp.int32)


_route = pl.pallas_call(
    _route_body,
    out_shape=(jax.ShapeDtypeStruct((1, S), jnp.float32),
               jax.ShapeDtypeStruct((S, H), jnp.float32),
               jax.ShapeDtypeStruct((S, 1), jnp.int32),
               jax.ShapeDtypeStruct((1, CAP), jnp.int32),
               jax.ShapeDtypeStruct((1, CAP), jnp.float32)),
)


def _kv_body(xn_ref, wk_ref, wv_ref, k_ref, v_ref):
    xn = xn_ref[...].astype(BF)
    k_ref[...] = jnp.dot(xn, wk_ref[...].astype(BF),
                         preferred_element_type=jnp.float32).astype(BF)
    v_ref[...] = jnp.dot(xn, wv_ref[...].astype(BF),
                         preferred_element_type=jnp.float32).astype(BF)


_kv = pl.pallas_call(
    _kv_body,
    grid=(4, 8),
    in_specs=[
        pl.BlockSpec((256, H), lambda j, i: (i, 0)),
        pl.BlockSpec((H, 512), lambda j, i: (0, j)),
        pl.BlockSpec((H, 512), lambda j, i: (0, j)),
    ],
    out_specs=[pl.BlockSpec((256, 512), lambda j, i: (i, j)),
               pl.BlockSpec((256, 512), lambda j, i: (i, j))],
    out_shape=[jax.ShapeDtypeStruct((S, H), BF),
               jax.ShapeDtypeStruct((S, H), BF)],
)


def _q_body(xs_ref, g1_ref, b1_ref, wq_ref, q_ref):
    xn = _ln(xs_ref[...], g1_ref[...], b1_ref[...]).astype(BF)
    q_ref[...] = jnp.dot(xn, wq_ref[...].astype(BF),
                         preferred_element_type=jnp.float32).astype(BF)


_q = pl.pallas_call(
    _q_body,
    grid=(4,),
    in_specs=[
        pl.BlockSpec((CAP, H), lambda j: (0, 0)),
        pl.BlockSpec((1, H), lambda j: (0, 0)),
        pl.BlockSpec((1, H), lambda j: (0, 0)),
        pl.BlockSpec((H, 512), lambda j: (0, j)),
    ],
    out_specs=pl.BlockSpec((CAP, 512), lambda j: (0, j)),
    out_shape=jax.ShapeDtypeStruct((CAP, H), BF),
)


def _attn_body(q_ref, k_ref, v_ref, pos_ref, o_ref):
    s = lax.dot_general(
        q_ref[...], k_ref[...], (((1,), (1,)), ((), ())),
        preferred_element_type=jnp.float32) * (DH ** -0.5)
    i0 = lax.broadcasted_iota(jnp.int32, (256, 256), 0)
    j1 = lax.broadcasted_iota(jnp.int32, (256, 256), 1)
    eye = (i0 == j1).astype(jnp.float32)
    pos_col = lax.dot_general(eye, pos_ref[...], (((1,), (1,)), ((), ())),
                              preferred_element_type=jnp.float32)
    kj = lax.broadcasted_iota(jnp.int32, (256, S), 1).astype(jnp.float32)
    s = jnp.where(kj <= pos_col, s, NEG)
    m = jnp.max(s, axis=1, keepdims=True)
    e = jnp.exp(s - m)
    p = (e / jnp.sum(e, axis=1, keepdims=True)).astype(BF)
    o_ref[...] = jnp.dot(p, v_ref[...],
                         preferred_element_type=jnp.float32).astype(BF)


_attn = pl.pallas_call(
    _attn_body,
    grid=(NH, 4),
    in_specs=[
        pl.BlockSpec((256, DH), lambda h, i: (i, h)),
        pl.BlockSpec((S, DH), lambda h, i: (0, h)),
        pl.BlockSpec((S, DH), lambda h, i: (0, h)),
        pl.BlockSpec((1, 256), lambda h, i: (0, i)),
    ],
    out_specs=pl.BlockSpec((256, DH), lambda h, i: (i, h)),
    out_shape=jax.ShapeDtypeStruct((CAP, H), BF),
)


def _oproj_body(ao_ref, wo_ref, xs_ref, g2_ref, b2_ref, h_ref, xn_ref):
    hb = xs_ref[...] + jnp.dot(ao_ref[...], wo_ref[...].astype(BF),
                               preferred_element_type=jnp.float32)
    h_ref[...] = hb
    xn_ref[...] = _ln(hb, g2_ref[...], b2_ref[...]).astype(BF)


_oproj = pl.pallas_call(
    _oproj_body,
    grid=(4,),
    in_specs=[
        pl.BlockSpec((256, H), lambda i: (i, 0)),
        pl.BlockSpec((H, H), lambda i: (0, 0)),
        pl.BlockSpec((256, H), lambda i: (i, 0)),
        pl.BlockSpec((1, H), lambda i: (0, 0)),
        pl.BlockSpec((1, H), lambda i: (0, 0)),
    ],
    out_specs=[pl.BlockSpec((256, H), lambda i: (i, 0)),
               pl.BlockSpec((256, H), lambda i: (i, 0))],
    out_shape=[jax.ShapeDtypeStruct((CAP, H), jnp.float32),
               jax.ShapeDtypeStruct((CAP, H), BF)],
)


def _ffn_body(xn_ref, h_ref, wg_ref, wu_ref, wd_ref, y_ref):
    j = pl.program_id(0)
    xn = xn_ref[...]
    g = jnp.dot(xn, wg_ref[...].astype(BF), preferred_element_type=jnp.float32)
    u = jnp.dot(xn, wu_ref[...].astype(BF), preferred_element_type=jnp.float32)
    f = (g / (1.0 + jnp.exp(-g)) * u).astype(BF)
    c = jnp.dot(f, wd_ref[...].astype(BF), preferred_element_type=jnp.float32)

    @pl.when(j == 0)
    def _():
        y_ref[...] = h_ref[...] + c

    @pl.when(j != 0)
    def _():
        y_ref[...] = y_ref[...] + c


_ffn = pl.pallas_call(
    _ffn_body,
    grid=(16,),
    in_specs=[
        pl.BlockSpec((CAP, H), lambda j: (0, 0)),
        pl.BlockSpec((CAP, H), lambda j: (0, 0)),
        pl.BlockSpec((H, 512), lambda j: (0, j)),
        pl.BlockSpec((H, 512), lambda j: (0, j)),
        pl.BlockSpec((512, H), lambda j: (j, 0)),
    ],
    out_specs=pl.BlockSpec((CAP, H), lambda j: (0, 0)),
    out_shape=jax.ShapeDtypeStruct((CAP, H), jnp.float32),
)


@functools.lru_cache(maxsize=None)
def _sc_gather_sel(chunk=32):
    info = plsc.get_sparse_core_info()
    nc, ns = info.num_cores, info.num_subcores
    b_per_w = CAP // (nc * ns)
    mesh = plsc.VectorSubcoreMesh(core_axis_name="c", subcore_axis_name="s")

    @functools.partial(
        pl.kernel, mesh=mesh,
        out_type=jax.ShapeDtypeStruct((CAP, H), jnp.float32),
        scratch_types=[
            pltpu.VMEM((chunk,), jnp.int32),
            pltpu.VMEM((chunk, H), jnp.float32),
            pltpu.SemaphoreType.DMA,
        ],
    )
    def k(x_hbm, idx_hbm, xs_hbm, idx_v, rows_v, sem):
        wid = lax.axis_index("s") * nc + lax.axis_index("c")
        base = wid * b_per_w
        for t in range(b_per_w // chunk):
            off = base + t * chunk
            pltpu.sync_copy(idx_hbm.at[pl.ds(off, chunk)], idx_v)
            pltpu.async_copy(x_hbm.at[idx_v], rows_v, sem).wait()
            pltpu.sync_copy(rows_v, xs_hbm.at[pl.ds(off, chunk)])

    return k


@functools.lru_cache(maxsize=None)
def _sc_gather_out(chunk=32):
    info = plsc.get_sparse_core_info()
    nc, ns = info.num_cores, info.num_subcores
    b_per_w = S // (nc * ns)
    mesh = plsc.VectorSubcoreMesh(core_axis_name="c", subcore_axis_name="s")

    @functools.partial(
        pl.kernel, mesh=mesh,
        out_type=jax.ShapeDtypeStruct((S, H), jnp.float32),
        scratch_types=[
            pltpu.VMEM((chunk,), jnp.int32),
            pltpu.VMEM((chunk, H), jnp.float32),
            pltpu.SemaphoreType.DMA,
        ],
    )
    def k(table_hbm, idx_hbm, out_hbm, idx_v, rows_v, sem):
        wid = lax.axis_index("s") * nc + lax.axis_index("c")
        base = wid * b_per_w
        for t in range(b_per_w // chunk):
            off = base + t * chunk
            pltpu.sync_copy(idx_hbm.at[pl.ds(off, chunk)], idx_v)
            pltpu.async_copy(table_hbm.at[idx_v], rows_v, sem).wait()
            pltpu.sync_copy(rows_v, out_hbm.at[pl.ds(off, chunk)])

    return k


def kernel(x, W_router, ln1_g, ln1_b, Wq, Wk, Wv, Wo, ln2_g, ln2_b, Wg, Wu, Wd):
    B = x.shape[0]
    x2 = x.reshape(S, H)
    wr_row = W_router.reshape(1, H)
    g1 = ln1_g.reshape(1, H)
    b1 = ln1_b.reshape(1, H)
    g2 = ln2_g.reshape(1, H)
    b2 = ln2_b.reshape(1, H)

    logits_row, xn, gidx_col, sel_row, pos_row = _route(x2, wr_row, g1, b1)
    sel = sel_row.reshape(CAP)
    gidx = gidx_col.reshape(S)

    x_sel = _sc_gather_sel()(x2, sel)
    k_all, v_all = _kv(xn, Wk, Wv)
    q = _q(x_sel, g1, b1, Wq)
    ao = _attn(q, k_all, v_all, pos_row)
    h_sel, xn2 = _oproj(ao, Wo, x_sel, g2, b2)
    y_sel = _ffn(xn2, h_sel, Wg, Wu, Wd)

    table = jnp.concatenate([x2, y_sel], axis=0)
    out2 = _sc_gather_out()(table, gidx)
    return (out2.reshape(B, S, H), logits_row.reshape(B, S))


# speedup vs baseline: 2.0188x; 1.0163x over previous
import functools

import jax
import jax.numpy as jnp
from jax import lax
from jax.experimental import pallas as pl
from jax.experimental.pallas import tpu as pltpu
from jax.experimental.pallas import tpu_sc as plsc

H = 2048
NH = 16
DH = H // NH
FF = 8192
CAP = 1024
S = 2048
EPS = 1e-6
NEG = -1e9
BF = jnp.bfloat16


def _ln(xb, g, b):
    m = jnp.mean(xb, axis=-1, keepdims=True)
    d = xb - m
    v = jnp.mean(d * d, axis=-1, keepdims=True)
    return d / jnp.sqrt(v + EPS) * g + b


def _route_body(x_ref, wr_ref, g1_ref, b1_ref, lg_ref, xn_ref, g_ref, idx_ref,
                posf_ref):
    x = x_ref[...]
    xn_ref[...] = _ln(x, g1_ref[...], b1_ref[...]).astype(BF)
    v_row = lax.dot_general(wr_ref[...], x, (((1,), (1,)), ((), ())),
                            preferred_element_type=jnp.float32)
    lg_ref[...] = v_row
    CH = 256
    i0 = lax.broadcasted_iota(jnp.int32, (S, CH), 0)
    v_col = jnp.zeros((S, 1), jnp.float32)
    for t in range(S // CH):
        j1 = lax.broadcasted_iota(jnp.int32, (S, CH), 1) + t * CH
        vjb = v_row[:, t * CH:(t + 1) * CH]
        eye_t = (i0 == j1).astype(jnp.float32)
        v_col = v_col + lax.dot_general(
            eye_t, vjb, (((1,), (1,)), ((), ())),
            preferred_element_type=jnp.float32)
    rank_col = jnp.zeros((S, 1), jnp.float32)
    slot_col = jnp.zeros((S, 1), jnp.float32)
    for t in range(S // CH):
        j1 = lax.broadcasted_iota(jnp.int32, (S, CH), 1) + t * CH
        vjb = v_row[:, t * CH:(t + 1) * CH]
        beats = (vjb > v_col) | ((vjb == v_col) & (j1 < i0))
        rank_col = rank_col + jnp.sum(
            jnp.where(beats, 1.0, 0.0), axis=1, keepdims=True)
        beaten = (v_col > vjb) | ((v_col == vjb) & (i0 < j1))
        rank_row_c = jnp.sum(jnp.where(beaten, 1.0, 0.0), axis=0, keepdims=True)
        m_row_c = jnp.where(rank_row_c < CAP, 1.0, 0.0)
        slot_col = slot_col + jnp.sum(
            jnp.where(j1 < i0, m_row_c * jnp.ones((S, CH), jnp.float32), 0.0),
            axis=1, keepdims=True)
    m_col = rank_col < CAP
    icol = lax.broadcasted_iota(jnp.int32, (S, 1), 0)
    sloti = slot_col.astype(jnp.int32)
    g_ref[...] = jnp.where(m_col, S + sloti, icol)
    i0f = i0.astype(jnp.float32)
    for u in range(CAP // CH):
        sc = lax.broadcasted_iota(jnp.int32, (S, CH), 1) + u * CH
        contrib = jnp.where(m_col & (sloti == sc), i0f, 0.0)
        idx_f = jnp.sum(contrib, axis=0, keepdims=True)
        posf_ref[:, u * CH:(u + 1) * CH] = idx_f
        idx_ref[:, u * CH:(u + 1) * CH] = idx_f.astype(jnp.int32)


_route = pl.pallas_call(
    _route_body,
    out_shape=(jax.ShapeDtypeStruct((1, S), jnp.float32),
               jax.ShapeDtypeStruct((S, H), BF),
               jax.ShapeDtypeStruct((S, 1), jnp.int32),
               jax.ShapeDtypeStruct((1, CAP), jnp.int32),
               jax.ShapeDtypeStruct((1, CAP), jnp.float32)),
)


def _kv_body(xn_ref, wk_ref, wv_ref, k_ref, v_ref, wkb_ref, wvb_ref):
    i = pl.program_id(1)

    @pl.when(i == 0)
    def _():
        wkb_ref[...] = wk_ref[...].astype(BF)
        wvb_ref[...] = wv_ref[...].astype(BF)

    xn = xn_ref[...]
    k_ref[...] = jnp.dot(xn, wkb_ref[...],
                         preferred_element_type=jnp.float32).astype(BF)
    v_ref[...] = jnp.dot(xn, wvb_ref[...],
                         preferred_element_type=jnp.float32).astype(BF)


_kv = pl.pallas_call(
    _kv_body,
    grid=(2, 8),
    in_specs=[
        pl.BlockSpec((256, H), lambda j, i: (i, 0)),
        pl.BlockSpec((H, 1024), lambda j, i: (0, j)),
        pl.BlockSpec((H, 1024), lambda j, i: (0, j)),
    ],
    out_specs=[pl.BlockSpec((256, 1024), lambda j, i: (i, j)),
               pl.BlockSpec((256, 1024), lambda j, i: (i, j))],
    out_shape=[jax.ShapeDtypeStruct((S, H), BF),
               jax.ShapeDtypeStruct((S, H), BF)],
    scratch_shapes=[pltpu.VMEM((H, 1024), BF), pltpu.VMEM((H, 1024), BF)],
)


def _q_body(xs_ref, g1_ref, b1_ref, wq_ref, q_ref):
    xn = _ln(xs_ref[...], g1_ref[...], b1_ref[...]).astype(BF)
    q_ref[...] = jnp.dot(xn, wq_ref[...].astype(BF),
                         preferred_element_type=jnp.float32).astype(BF)


_q = pl.pallas_call(
    _q_body,
    grid=(4,),
    in_specs=[
        pl.BlockSpec((CAP, H), lambda j: (0, 0)),
        pl.BlockSpec((1, H), lambda j: (0, 0)),
        pl.BlockSpec((1, H), lambda j: (0, 0)),
        pl.BlockSpec((H, 512), lambda j: (0, j)),
    ],
    out_specs=pl.BlockSpec((CAP, 512), lambda j: (0, j)),
    out_shape=jax.ShapeDtypeStruct((CAP, H), BF),
)


def _attn_body(q_ref, k_ref, v_ref, pos_ref, o_ref):
    s = lax.dot_general(
        q_ref[...], k_ref[...], (((1,), (1,)), ((), ())),
        preferred_element_type=jnp.float32) * (DH ** -0.5)
    i0 = lax.broadcasted_iota(jnp.int32, (256, 256), 0)
    j1 = lax.broadcasted_iota(jnp.int32, (256, 256), 1)
    eye = (i0 == j1).astype(jnp.float32)
    pos_col = lax.dot_general(eye, pos_ref[...], (((1,), (1,)), ((), ())),
                              preferred_element_type=jnp.float32)
    kj = lax.broadcasted_iota(jnp.int32, (256, S), 1).astype(jnp.float32)
    s = jnp.where(kj <= pos_col, s, NEG)
    m = jnp.max(s, axis=1, keepdims=True)
    e = jnp.exp(s - m)
    p = (e / jnp.sum(e, axis=1, keepdims=True)).astype(BF)
    o_ref[...] = jnp.dot(p, v_ref[...],
                         preferred_element_type=jnp.float32).astype(BF)


_attn = pl.pallas_call(
    _attn_body,
    grid=(NH, 4),
    in_specs=[
        pl.BlockSpec((256, DH), lambda h, i: (i, h)),
        pl.BlockSpec((S, DH), lambda h, i: (0, h)),
        pl.BlockSpec((S, DH), lambda h, i: (0, h)),
        pl.BlockSpec((1, 256), lambda h, i: (0, i)),
    ],
    out_specs=pl.BlockSpec((256, DH), lambda h, i: (i, h)),
    out_shape=jax.ShapeDtypeStruct((CAP, H), BF),
)


def _oproj_body(ao_ref, wo_ref, xs_ref, g2_ref, b2_ref, h_ref, xn_ref):
    hb = xs_ref[...] + jnp.dot(ao_ref[...], wo_ref[...].astype(BF),
                               preferred_element_type=jnp.float32)
    h_ref[...] = hb
    xn_ref[...] = _ln(hb, g2_ref[...], b2_ref[...]).astype(BF)


_oproj = pl.pallas_call(
    _oproj_body,
    grid=(4,),
    in_specs=[
        pl.BlockSpec((256, H), lambda i: (i, 0)),
        pl.BlockSpec((H, H), lambda i: (0, 0)),
        pl.BlockSpec((256, H), lambda i: (i, 0)),
        pl.BlockSpec((1, H), lambda i: (0, 0)),
        pl.BlockSpec((1, H), lambda i: (0, 0)),
    ],
    out_specs=[pl.BlockSpec((256, H), lambda i: (i, 0)),
               pl.BlockSpec((256, H), lambda i: (i, 0))],
    out_shape=[jax.ShapeDtypeStruct((CAP, H), jnp.float32),
               jax.ShapeDtypeStruct((CAP, H), BF)],
)


def _ffn_body(xn_ref, h_ref, wg_ref, wu_ref, wd_ref, y_ref):
    j = pl.program_id(0)
    xn = xn_ref[...]
    g = jnp.dot(xn, wg_ref[...].astype(BF), preferred_element_type=jnp.float32)
    u = jnp.dot(xn, wu_ref[...].astype(BF), preferred_element_type=jnp.float32)
    f = (g / (1.0 + jnp.exp(-g)) * u).astype(BF)
    c = jnp.dot(f, wd_ref[...].astype(BF), preferred_element_type=jnp.float32)

    @pl.when(j == 0)
    def _():
        y_ref[...] = h_ref[...] + c

    @pl.when(j != 0)
    def _():
        y_ref[...] = y_ref[...] + c


_ffn = pl.pallas_call(
    _ffn_body,
    grid=(16,),
    in_specs=[
        pl.BlockSpec((CAP, H), lambda j: (0, 0)),
        pl.BlockSpec((CAP, H), lambda j: (0, 0)),
        pl.BlockSpec((H, 512), lambda j: (0, j)),
        pl.BlockSpec((H, 512), lambda j: (0, j)),
        pl.BlockSpec((512, H), lambda j: (j, 0)),
    ],
    out_specs=pl.BlockSpec((CAP, H), lambda j: (0, 0)),
    out_shape=jax.ShapeDtypeStruct((CAP, H), jnp.float32),
)


@functools.lru_cache(maxsize=None)
def _sc_gather_sel(chunk=32):
    info = plsc.get_sparse_core_info()
    nc, ns = info.num_cores, info.num_subcores
    b_per_w = CAP // (nc * ns)
    mesh = plsc.VectorSubcoreMesh(core_axis_name="c", subcore_axis_name="s")

    @functools.partial(
        pl.kernel, mesh=mesh,
        out_type=jax.ShapeDtypeStruct((CAP, H), jnp.float32),
        scratch_types=[
            pltpu.VMEM((chunk,), jnp.int32),
            pltpu.VMEM((chunk, H), jnp.float32),
            pltpu.SemaphoreType.DMA,
        ],
    )
    def k(x_hbm, idx_hbm, xs_hbm, idx_v, rows_v, sem):
        wid = lax.axis_index("s") * nc + lax.axis_index("c")
        base = wid * b_per_w
        for t in range(b_per_w // chunk):
            off = base + t * chunk
            pltpu.sync_copy(idx_hbm.at[pl.ds(off, chunk)], idx_v)
            pltpu.async_copy(x_hbm.at[idx_v], rows_v, sem).wait()
            pltpu.sync_copy(rows_v, xs_hbm.at[pl.ds(off, chunk)])

    return k


@functools.lru_cache(maxsize=None)
def _sc_gather_out(chunk=32):
    info = plsc.get_sparse_core_info()
    nc, ns = info.num_cores, info.num_subcores
    b_per_w = S // (nc * ns)
    mesh = plsc.VectorSubcoreMesh(core_axis_name="c", subcore_axis_name="s")

    @functools.partial(
        pl.kernel, mesh=mesh,
        out_type=jax.ShapeDtypeStruct((S, H), jnp.float32),
        scratch_types=[
            pltpu.VMEM((chunk,), jnp.int32),
            pltpu.VMEM((chunk, H), jnp.float32),
            pltpu.SemaphoreType.DMA,
        ],
    )
    def k(table_hbm, idx_hbm, out_hbm, idx_v, rows_v, sem):
        wid = lax.axis_index("s") * nc + lax.axis_index("c")
        base = wid * b_per_w
        for t in range(b_per_w // chunk):
            off = base + t * chunk
            pltpu.sync_copy(idx_hbm.at[pl.ds(off, chunk)], idx_v)
            pltpu.async_copy(table_hbm.at[idx_v], rows_v, sem).wait()
            pltpu.sync_copy(rows_v, out_hbm.at[pl.ds(off, chunk)])

    return k


def kernel(x, W_router, ln1_g, ln1_b, Wq, Wk, Wv, Wo, ln2_g, ln2_b, Wg, Wu, Wd):
    B = x.shape[0]
    x2 = x.reshape(S, H)
    wr_row = W_router.reshape(1, H)
    g1 = ln1_g.reshape(1, H)
    b1 = ln1_b.reshape(1, H)
    g2 = ln2_g.reshape(1, H)
    b2 = ln2_b.reshape(1, H)

    logits_row, xn, gidx_col, sel_row, pos_row = _route(x2, wr_row, g1, b1)
    sel = sel_row.reshape(CAP)
    gidx = gidx_col.reshape(S)

    x_sel = _sc_gather_sel()(x2, sel)
    k_all, v_all = _kv(xn, Wk, Wv)
    q = _q(x_sel, g1, b1, Wq)
    ao = _attn(q, k_all, v_all, pos_row)
    h_sel, xn2 = _oproj(ao, Wo, x_sel, g2, b2)
    y_sel = _ffn(xn2, h_sel, Wg, Wu, Wd)

    table = jnp.concatenate([x2, y_sel], axis=0)
    out2 = _sc_gather_out()(table, gidx)
    return (out2.reshape(B, S, H), logits_row.reshape(B, S))
